# baseline (speedup 1.0000x reference)
import jax
import jax.numpy as jnp
from jax.experimental import pallas as pl
from jax.experimental.pallas import tpu as pltpu

N_FFT = 1024
HOP = 256
WIN = 1024
EPS = 1e-11
B = 8
K = 513
T = 2048
FB = 512
NBF = T // FB
NB = NBF + 1


def _weights():
    n = jnp.arange(WIN, dtype=jnp.float32)[:, None]
    k = jnp.arange(K, dtype=jnp.float32)[None, :]
    scale = jnp.where((k == 0) | (k == K - 1), 1.0, 2.0) / N_FFT
    c = jnp.cos(2.0 * jnp.pi * n * k / N_FFT) * scale
    w = jnp.hanning(WIN).astype(jnp.float32)
    wc = w[:, None] * c
    w2 = (w * w).reshape(4, HOP)
    m = jnp.arange(8)[:, None]
    m = jnp.where(m >= 4, m - 4 + T, m)
    j = jnp.arange(4)[None, :]
    valid = ((m >= j) & (m - j < T)).astype(jnp.float32)
    env = jnp.einsum("mj,jr->mr", valid, w2)
    return wc.astype(jnp.bfloat16), 1.0 / (env + EPS)


def _shift(c, j):
    parts = []
    if j:
        parts.append(jnp.zeros((j, HOP), jnp.float32))
    parts.append(c)
    if 3 - j:
        parts.append(jnp.zeros((3 - j, HOP), jnp.float32))
    return jnp.concatenate(parts, axis=0)


def _istft_block(spec_ref, wc_ref, env_ref, out_ref, prev_ref, carry_ref,
                 norm_ref):
    i = pl.program_id(1)
    e_full = env_ref[3]

    @pl.when(i < NBF)
    def _compute():
        s = spec_ref[0].astype(jnp.bfloat16)
        f = jax.lax.dot_general(
            s, wc_ref[...], (((0,), (1,)), ((), ())),
            preferred_element_type=jnp.float32)
        acc = jnp.zeros((FB + 3, HOP), jnp.float32)
        for j in range(4):
            acc = acc + _shift(f[:, HOP * j:HOP * (j + 1)], j)
        prev_carry = jnp.where(i > 0, carry_ref[...], 0.0)
        top3 = acc[:3] + prev_carry
        e3 = jnp.where(i == 0, env_ref[:3],
                       jnp.broadcast_to(e_full, (3, HOP)))
        norm_ref[...] = jnp.concatenate(
            [top3 * e3, acc[3:FB] * e_full], axis=0)
        carry_ref[...] = acc[FB:]

    @pl.when(i == NBF)
    def _flush():
        norm_ref[:3] = carry_ref[...] * env_ref[4:7]

    norm = norm_ref[...]
    chunks = jnp.concatenate([prev_ref[1:], norm[:2]], axis=0)
    out_ref[0, 0] = jnp.concatenate(
        [chunks[:FB, HOP // 2:], chunks[1:, :HOP // 2]], axis=1)
    prev_ref[...] = norm


def kernel(spec):
    wc, env = _weights()
    out = pl.pallas_call(
        _istft_block,
        grid=(B, NB),
        in_specs=[
            pl.BlockSpec((1, K, FB), lambda b, i: (b, 0, jnp.minimum(i, NBF - 1))),
            pl.BlockSpec((WIN, K), lambda b, i: (0, 0)),
            pl.BlockSpec((8, HOP), lambda b, i: (0, 0)),
        ],
        out_specs=pl.BlockSpec(
            (1, 1, FB, HOP), lambda b, i: (b, jnp.maximum(i - 1, 0), 0, 0)),
        out_shape=jax.ShapeDtypeStruct((B, NBF, FB, HOP), jnp.float32),
        scratch_shapes=[pltpu.VMEM((FB, HOP), jnp.float32),
                        pltpu.VMEM((3, HOP), jnp.float32),
                        pltpu.VMEM((FB, HOP), jnp.float32)],
    )(spec, wc, env)
    return out.reshape(B, NBF * FB * HOP)


# speedup vs baseline: 1.1855x; 1.1855x over previous
import jax
import jax.numpy as jnp
from jax.experimental import pallas as pl
from jax.experimental.pallas import tpu as pltpu

N_FFT = 1024
HOP = 256
WIN = 1024
EPS = 1e-11
B = 8
K = 513
T = 2048
FB = 512
NBF = T // FB


def _weights():
    n = jnp.arange(WIN, dtype=jnp.float32)[:, None]
    k = jnp.arange(K, dtype=jnp.float32)[None, :]
    scale = jnp.where((k == 0) | (k == K - 1), 1.0, 2.0) / N_FFT
    c = jnp.cos(2.0 * jnp.pi * n * k / N_FFT) * scale
    w = jnp.hanning(WIN).astype(jnp.float32)
    wc = w[:, None] * c
    w2 = (w * w).reshape(4, HOP)
    m = jnp.arange(8)[:, None]
    m = jnp.where(m >= 4, m - 4 + T, m)
    j = jnp.arange(4)[None, :]
    valid = ((m >= j) & (m - j < T)).astype(jnp.float32)
    env = jnp.einsum("mj,jr->mr", valid, w2)
    return wc.astype(jnp.bfloat16), 1.0 / (env + EPS)


def _shift(c, j):
    parts = []
    if j:
        parts.append(jnp.zeros((j, HOP), jnp.float32))
    parts.append(c)
    if 3 - j:
        parts.append(jnp.zeros((3 - j, HOP), jnp.float32))
    return jnp.concatenate(parts, axis=0)


def _istft_batch(spec_ref, wc_ref, env_ref, out_ref):
    e_full = env_ref[3]
    prev = None
    carry = None
    for i in range(NBF):
        s = spec_ref[0, :, FB * i:FB * (i + 1)].astype(jnp.bfloat16)
        f = jax.lax.dot_general(
            s, wc_ref[...], (((0,), (1,)), ((), ())),
            preferred_element_type=jnp.float32)
        acc = jnp.zeros((FB + 3, HOP), jnp.float32)
        for j in range(4):
            acc = acc + _shift(f[:, HOP * j:HOP * (j + 1)], j)
        if i == 0:
            norm = jnp.concatenate(
                [acc[:3] * env_ref[:3], acc[3:FB] * e_full], axis=0)
        else:
            norm = jnp.concatenate(
                [(acc[:3] + carry) * e_full, acc[3:FB] * e_full], axis=0)
        carry = acc[FB:]
        if i > 0:
            chunks = jnp.concatenate([prev[1:], norm[:2]], axis=0)
            out_ref[0, i - 1] = jnp.concatenate(
                [chunks[:FB, HOP // 2:], chunks[1:, :HOP // 2]], axis=1)
        prev = norm
    tail = carry[:2] * env_ref[4:6]
    chunks = jnp.concatenate([prev[1:], tail], axis=0)
    out_ref[0, NBF - 1] = jnp.concatenate(
        [chunks[:FB, HOP // 2:], chunks[1:, :HOP // 2]], axis=1)


def kernel(spec):
    wc, env = _weights()
    out = pl.pallas_call(
        _istft_batch,
        grid=(B,),
        in_specs=[
            pl.BlockSpec((1, K, T), lambda b: (b, 0, 0)),
            pl.BlockSpec((WIN, K), lambda b: (0, 0)),
            pl.BlockSpec((8, HOP), lambda b: (0, 0)),
        ],
        out_specs=pl.BlockSpec(
            (1, NBF, FB, HOP), lambda b: (b, 0, 0, 0)),
        out_shape=jax.ShapeDtypeStruct((B, NBF, FB, HOP), jnp.float32),
    )(spec, wc, env)
    return out.reshape(B, NBF * FB * HOP)


# speedup vs baseline: 1.3288x; 1.1209x over previous
import jax
import jax.numpy as jnp
from jax.experimental import pallas as pl
from jax.experimental.pallas import tpu as pltpu

N_FFT = 1024
HOP = 256
WIN = 1024
EPS = 1e-11
B = 8
K = 513
T = 2048
FB = 512
NBF = T // FB


def _weights():
    n = jnp.arange(WIN, dtype=jnp.float32)[:, None]
    k = jnp.arange(K, dtype=jnp.float32)[None, :]
    scale = jnp.where((k == 0) | (k == K - 1), 1.0, 2.0) / N_FFT
    c = jnp.cos(2.0 * jnp.pi * n * k / N_FFT) * scale
    w = jnp.hanning(WIN).astype(jnp.float32)
    wc = w[:, None] * c
    w2 = (w * w).reshape(4, HOP)
    m = jnp.arange(8)[:, None]
    m = jnp.where(m >= 4, m - 4 + T, m)
    j = jnp.arange(4)[None, :]
    valid = ((m >= j) & (m - j < T)).astype(jnp.float32)
    env = jnp.einsum("mj,jr->mr", valid, w2)
    return wc.astype(jnp.bfloat16), 1.0 / (env + EPS)


def _shift(c, j):
    parts = []
    if j:
        parts.append(jnp.zeros((j, HOP), jnp.float32))
    parts.append(c)
    if 3 - j:
        parts.append(jnp.zeros((3 - j, HOP), jnp.float32))
    return jnp.concatenate(parts, axis=0)


def _istft_batch(spec_ref, wc_ref, env_ref, out_ref):
    e_full = env_ref[3]
    prev = None
    carry = None
    for i in range(NBF):
        s = spec_ref[0, :, FB * i:FB * (i + 1)].astype(jnp.bfloat16)
        f = jax.lax.dot_general(
            s, wc_ref[...], (((0,), (1,)), ((), ())),
            preferred_element_type=jnp.float32)
        acc = jnp.zeros((FB + 3, HOP), jnp.float32)
        for j in range(4):
            acc = acc + _shift(f[:, HOP * j:HOP * (j + 1)], j)
        if i == 0:
            norm = jnp.concatenate(
                [acc[:3] * env_ref[:3], acc[3:FB] * e_full], axis=0)
        else:
            norm = jnp.concatenate(
                [(acc[:3] + carry) * e_full, acc[3:FB] * e_full], axis=0)
        carry = acc[FB:]
        if i > 0:
            chunks = jnp.concatenate([prev[1:], norm[:2]], axis=0)
            out_ref[0, i - 1] = jnp.concatenate(
                [chunks[:FB, HOP // 2:], chunks[1:, :HOP // 2]], axis=1)
        prev = norm
    tail = carry[:2] * env_ref[4:6]
    chunks = jnp.concatenate([prev[1:], tail], axis=0)
    out_ref[0, NBF - 1] = jnp.concatenate(
        [chunks[:FB, HOP // 2:], chunks[1:, :HOP // 2]], axis=1)


def _relayout(x_ref, o_ref):
    o_ref[...] = x_ref[...].reshape(B, -1)


def kernel(spec):
    wc, env = _weights()
    out = pl.pallas_call(
        _istft_batch,
        grid=(B,),
        in_specs=[
            pl.BlockSpec((1, K, T), lambda b: (b, 0, 0)),
            pl.BlockSpec((WIN, K), lambda b: (0, 0)),
            pl.BlockSpec((8, HOP), lambda b: (0, 0)),
        ],
        out_specs=pl.BlockSpec(
            (1, NBF, FB, HOP), lambda b: (b, 0, 0, 0)),
        out_shape=jax.ShapeDtypeStruct((B, NBF, FB, HOP), jnp.float32),
    )(spec, wc, env)
    y3 = out.reshape(B, T, HOP)
    TBR = 128
    out2 = pl.pallas_call(
        _relayout,
        grid=(T // TBR,),
        in_specs=[pl.BlockSpec((B, TBR, HOP), lambda g: (0, g, 0))],
        out_specs=pl.BlockSpec((B, TBR * HOP), lambda g: (0, g)),
        out_shape=jax.ShapeDtypeStruct((B, T * HOP), jnp.float32),
    )(y3)
    return out2


# speedup vs baseline: 1.3987x; 1.0526x over previous
import jax
import jax.numpy as jnp
from jax.experimental import pallas as pl
from jax.experimental.pallas import tpu as pltpu

N_FFT = 1024
HOP = 256
WIN = 1024
EPS = 1e-11
B = 8
K = 513
T = 2048
FB = 512
NBF = T // FB


def _weights():
    n = jnp.arange(WIN, dtype=jnp.float32)[:, None]
    k = jnp.arange(K, dtype=jnp.float32)[None, :]
    scale = jnp.where((k == 0) | (k == K - 1), 1.0, 2.0) / N_FFT
    c = jnp.cos(2.0 * jnp.pi * n * k / N_FFT) * scale
    w = jnp.hanning(WIN).astype(jnp.float32)
    wc = w[:, None] * c
    w2 = (w * w).reshape(4, HOP)
    m = jnp.arange(8)[:, None]
    m = jnp.where(m >= 4, m - 4 + T, m)
    j = jnp.arange(4)[None, :]
    valid = ((m >= j) & (m - j < T)).astype(jnp.float32)
    env = jnp.einsum("mj,jr->mr", valid, w2)
    return wc.astype(jnp.bfloat16), 1.0 / (env + EPS)


def _shift(c, j):
    parts = []
    if j:
        parts.append(jnp.zeros((j, HOP), jnp.float32))
    parts.append(c)
    if 3 - j:
        parts.append(jnp.zeros((3 - j, HOP), jnp.float32))
    return jnp.concatenate(parts, axis=0)


def _istft_batch(spec_ref, wc_ref, env_ref, out_ref):
    e_full = env_ref[3]
    prev = None
    carry = None
    for i in range(NBF):
        s = spec_ref[0, :, FB * i:FB * (i + 1)].astype(jnp.bfloat16)
        f = jax.lax.dot_general(
            s, wc_ref[...], (((0,), (1,)), ((), ())),
            preferred_element_type=jnp.float32)
        acc = jnp.zeros((FB + 3, HOP), jnp.float32)
        for j in range(4):
            acc = acc + _shift(f[:, HOP * j:HOP * (j + 1)], j)
        if i == 0:
            norm = jnp.concatenate(
                [acc[:3] * env_ref[:3], acc[3:FB] * e_full], axis=0)
        else:
            norm = jnp.concatenate(
                [(acc[:3] + carry) * e_full, acc[3:FB] * e_full], axis=0)
        carry = acc[FB:]
        if i > 0:
            chunks = jnp.concatenate([prev[1:], norm[:2]], axis=0)
            out_ref[0, i - 1] = jnp.concatenate(
                [chunks[:FB, HOP // 2:], chunks[1:, :HOP // 2]], axis=1)
        prev = norm
    tail = carry[:2] * env_ref[4:6]
    chunks = jnp.concatenate([prev[1:], tail], axis=0)
    out_ref[0, NBF - 1] = jnp.concatenate(
        [chunks[:FB, HOP // 2:], chunks[1:, :HOP // 2]], axis=1)


def _relayout(x_ref, o_ref):
    o_ref[...] = x_ref[...].reshape(B, -1)


def kernel(spec):
    wc, env = _weights()
    out = pl.pallas_call(
        _istft_batch,
        grid=(B,),
        in_specs=[
            pl.BlockSpec((1, K, T), lambda b: (b, 0, 0)),
            pl.BlockSpec((WIN, K), lambda b: (0, 0)),
            pl.BlockSpec((8, HOP), lambda b: (0, 0)),
        ],
        out_specs=pl.BlockSpec(
            (1, NBF, FB, HOP), lambda b: (b, 0, 0, 0)),
        out_shape=jax.ShapeDtypeStruct((B, NBF, FB, HOP), jnp.float32),
    )(spec, wc, env)
    y3 = out.reshape(B, T, HOP)
    TBR = 256
    out2 = pl.pallas_call(
        _relayout,
        grid=(T // TBR,),
        in_specs=[pl.BlockSpec((B, TBR, HOP), lambda g: (0, g, 0))],
        out_specs=pl.BlockSpec((B, TBR * HOP), lambda g: (0, g)),
        out_shape=jax.ShapeDtypeStruct((B, T * HOP), jnp.float32),
    )(y3)
    return out2
